# baseline (speedup 1.0000x reference)
import functools
import math

import jax
import jax.numpy as jnp
from jax.experimental import pallas as pl
from jax.experimental.pallas import tpu as pltpu
from jax.experimental.pallas import tpu_sc as plsc

EPS = 1e-8

_SC_ROWS = 0

_SC_CORES = 2
_SC_SUBCORES = 16
_SC_WORKERS = _SC_CORES * _SC_SUBCORES
_SC_LANES = 16


def _make_sc_kernel(n, skip, nk):
    chunk = n // _SC_WORKERS
    nsteps = chunk // _SC_LANES
    mesh = plsc.VectorSubcoreMesh(core_axis_name="c", subcore_axis_name="s")

    @functools.partial(
        pl.kernel,
        mesh=mesh,
        out_type=jax.ShapeDtypeStruct((n,), jnp.float32),
        scratch_types=[
            pltpu.VMEM((chunk,), jnp.float32),
            pltpu.VMEM((chunk,), jnp.float32),
            pltpu.VMEM((nk,), jnp.float32),
            pltpu.VMEM((nk,), jnp.float32),
            pltpu.VMEM((nk,), jnp.float32),
            pltpu.VMEM((nk,), jnp.float32),
            pltpu.VMEM((_SC_LANES,), jnp.float32),
        ],
    )
    def sck(w_hbm, a_hbm, b_hbm, c0_hbm, mus_hbm, misc_hbm, out_hbm,
            w_v, out_v, a_v, b_v, c0_v, mus_v, misc_v):
        wid = jax.lax.axis_index("s") * _SC_CORES + jax.lax.axis_index("c")
        base = wid * chunk
        pltpu.sync_copy(w_hbm.at[pl.ds(skip + base, chunk)], w_v)
        pltpu.sync_copy(a_hbm, a_v)
        pltpu.sync_copy(b_hbm, b_v)
        pltpu.sync_copy(c0_hbm, c0_v)
        pltpu.sync_copy(mus_hbm, mus_v)
        pltpu.sync_copy(misc_hbm, misc_v)

        a_c = [a_v[pl.ds(h * _SC_LANES, _SC_LANES)] for h in range(nk // _SC_LANES)]
        b_c = [b_v[pl.ds(h * _SC_LANES, _SC_LANES)] for h in range(nk // _SC_LANES)]
        c0_c = [c0_v[pl.ds(h * _SC_LANES, _SC_LANES)] for h in range(nk // _SC_LANES)]
        mus_c = [mus_v[pl.ds(h * _SC_LANES, _SC_LANES)] for h in range(nk // _SC_LANES)]
        inv_t = misc_v[pl.ds(0, _SC_LANES)][0]

        def body(i, carry):
            off = i * _SC_LANES
            w = w_v[pl.ds(off, _SC_LANES)]
            w2 = w * w
            s = jnp.zeros((_SC_LANES,), jnp.float32)
            es = []
            for k in range(nk):
                h, j = divmod(k, _SC_LANES)
                e = jnp.exp(a_c[h][j] * w2 + (b_c[h][j] * w + c0_c[h][j]))
                es.append(e)
                s = s + e
            c = inv_t / (s + EPS)
            denom = jnp.zeros((_SC_LANES,), jnp.float32)
            num = jnp.zeros((_SC_LANES,), jnp.float32)
            for k in range(nk):
                h, j = divmod(k, _SC_LANES)
                p = jnp.exp(es[k] * c - 83.0)
                denom = denom + p
                num = num + p * mus_c[h][j]
            out_v[pl.ds(off, _SC_LANES)] = num / denom
            return carry

        jax.lax.fori_loop(0, nsteps, body, 0)
        pltpu.sync_copy(out_v, out_hbm.at[pl.ds(base, chunk)])

    return sck


def _gmm_tc_body(w_ref, pis_ref, mus_ref, sig_ref, t_ref, out_ref):
    w = w_ref[...]
    pis = jnp.abs(pis_ref[...])
    pi_norm = pis / jnp.sum(pis)
    sig = sig_ref[...]
    mus = mus_ref[...]
    sig2 = sig * sig
    log2e = 1.4426950408889634
    a = (-0.5 * log2e) / sig2
    b = -2.0 * a * mus
    c0 = a * mus * mus + (jnp.log(pi_norm) - 0.5 * jnp.log(2.0 * math.pi * sig2)) * log2e

    w2 = w * w
    le = a[:, :, None] * w2[None, :, :] + (b[:, :, None] * w[None, :, :] + c0[:, :, None])
    e = jnp.exp2(le)
    s = jnp.sum(e, axis=0)
    c = log2e / (t_ref[0, 0] * (s + EPS))
    p = jnp.exp2(e * c[None, :, :] - 121.0)
    denom = jnp.sum(p, axis=0)
    num = jnp.sum(p * mus[:, :, None], axis=0)
    out_ref[...] = num / denom


def _tc_call(weights, tc_rows, pis, mus, sigmas, temp):
    R, C = weights.shape
    K = pis.shape[0]
    B = 32
    return pl.pallas_call(
        _gmm_tc_body,
        grid=(tc_rows // B,),
        in_specs=[
            pl.BlockSpec((B, C), lambda i: (i, 0)),
            pl.BlockSpec((K, 1), lambda i: (0, 0)),
            pl.BlockSpec((K, 1), lambda i: (0, 0)),
            pl.BlockSpec((K, 1), lambda i: (0, 0)),
            pl.BlockSpec((1, 1), lambda i: (0, 0)),
        ],
        out_specs=pl.BlockSpec((B, C), lambda i: (i, 0)),
        out_shape=jax.ShapeDtypeStruct((tc_rows, C), weights.dtype),
        compiler_params=pltpu.CompilerParams(
            dimension_semantics=("parallel",),
            vmem_limit_bytes=100663296),
    )(weights, pis, mus, sigmas, temp)


def kernel(weights, mu, pi_k, pi_zero, sigma, sigma_zero, temperature):
    K = mu.shape[0] + 1
    R, C = weights.shape
    pis = jnp.concatenate([pi_zero, pi_k]).reshape(K, 1)
    mus = jnp.concatenate([jnp.zeros((1,), weights.dtype), mu]).reshape(K, 1)
    sigmas = jnp.concatenate([sigma_zero, sigma]).reshape(K, 1)
    temp = temperature.reshape(1, 1)

    tc_rows = R - _SC_ROWS
    outs = []
    if tc_rows > 0:
        outs.append(_tc_call(weights, tc_rows, pis, mus, sigmas, temp))
    if _SC_ROWS > 0:
        pi_n = jnp.abs(pis[:, 0])
        pi_n = pi_n / jnp.sum(pi_n)
        sc_sig2 = sigmas[:, 0] * sigmas[:, 0]
        sc_a = -0.5 / sc_sig2
        sc_mus = mus[:, 0]
        sc_b = -2.0 * sc_a * sc_mus
        sc_c0 = sc_a * sc_mus * sc_mus + jnp.log(pi_n) - 0.5 * jnp.log(
            2.0 * math.pi * sc_sig2)
        misc = jnp.zeros((_SC_LANES,), jnp.float32).at[0].set(
            1.0 / temperature[0])
        w_flat = weights.reshape(-1)
        n_sc = _SC_ROWS * C
        sck = _make_sc_kernel(n_sc, tc_rows * C, K)
        out_sc = sck(w_flat, sc_a, sc_b, sc_c0, sc_mus, misc)
        outs.append(out_sc.reshape(_SC_ROWS, C))
    if len(outs) == 1:
        return outs[0]
    return jnp.concatenate(outs, axis=0)


# speedup vs baseline: 1.0991x; 1.0991x over previous
import functools
import math

import jax
import jax.numpy as jnp
from jax.experimental import pallas as pl
from jax.experimental.pallas import tpu as pltpu
from jax.experimental.pallas import tpu_sc as plsc

EPS = 1e-8

_SC_ROWS = 0

_SC_CORES = 2
_SC_SUBCORES = 16
_SC_WORKERS = _SC_CORES * _SC_SUBCORES
_SC_LANES = 16


def _make_sc_kernel(n, skip, nk):
    chunk = n // _SC_WORKERS
    nsteps = chunk // _SC_LANES
    mesh = plsc.VectorSubcoreMesh(core_axis_name="c", subcore_axis_name="s")

    @functools.partial(
        pl.kernel,
        mesh=mesh,
        out_type=jax.ShapeDtypeStruct((n,), jnp.float32),
        scratch_types=[
            pltpu.VMEM((chunk,), jnp.float32),
            pltpu.VMEM((chunk,), jnp.float32),
            pltpu.VMEM((nk,), jnp.float32),
            pltpu.VMEM((nk,), jnp.float32),
            pltpu.VMEM((nk,), jnp.float32),
            pltpu.VMEM((nk,), jnp.float32),
            pltpu.VMEM((_SC_LANES,), jnp.float32),
        ],
    )
    def sck(w_hbm, a_hbm, b_hbm, c0_hbm, mus_hbm, misc_hbm, out_hbm,
            w_v, out_v, a_v, b_v, c0_v, mus_v, misc_v):
        wid = jax.lax.axis_index("s") * _SC_CORES + jax.lax.axis_index("c")
        base = wid * chunk
        pltpu.sync_copy(w_hbm.at[pl.ds(skip + base, chunk)], w_v)
        pltpu.sync_copy(a_hbm, a_v)
        pltpu.sync_copy(b_hbm, b_v)
        pltpu.sync_copy(c0_hbm, c0_v)
        pltpu.sync_copy(mus_hbm, mus_v)
        pltpu.sync_copy(misc_hbm, misc_v)

        a_c = [a_v[pl.ds(h * _SC_LANES, _SC_LANES)] for h in range(nk // _SC_LANES)]
        b_c = [b_v[pl.ds(h * _SC_LANES, _SC_LANES)] for h in range(nk // _SC_LANES)]
        c0_c = [c0_v[pl.ds(h * _SC_LANES, _SC_LANES)] for h in range(nk // _SC_LANES)]
        mus_c = [mus_v[pl.ds(h * _SC_LANES, _SC_LANES)] for h in range(nk // _SC_LANES)]
        inv_t = misc_v[pl.ds(0, _SC_LANES)][0]

        def body(i, carry):
            off = i * _SC_LANES
            w = w_v[pl.ds(off, _SC_LANES)]
            w2 = w * w
            s = jnp.zeros((_SC_LANES,), jnp.float32)
            es = []
            for k in range(nk):
                h, j = divmod(k, _SC_LANES)
                e = jnp.exp(a_c[h][j] * w2 + (b_c[h][j] * w + c0_c[h][j]))
                es.append(e)
                s = s + e
            c = inv_t / (s + EPS)
            denom = jnp.zeros((_SC_LANES,), jnp.float32)
            num = jnp.zeros((_SC_LANES,), jnp.float32)
            for k in range(nk):
                h, j = divmod(k, _SC_LANES)
                p = jnp.exp(es[k] * c - 83.0)
                denom = denom + p
                num = num + p * mus_c[h][j]
            out_v[pl.ds(off, _SC_LANES)] = num / denom
            return carry

        jax.lax.fori_loop(0, nsteps, body, 0)
        pltpu.sync_copy(out_v, out_hbm.at[pl.ds(base, chunk)])

    return sck


def _gmm_tc_body(w_ref, pis_ref, mus_ref, sig_ref, t_ref, out_ref):
    w = w_ref[...]
    pis = jnp.abs(pis_ref[...])
    pi_norm = pis / jnp.sum(pis)
    sig = sig_ref[...]
    mus = mus_ref[...]
    sig2 = sig * sig
    log2e = 1.4426950408889634
    a = (-0.5 * log2e) / sig2
    b = -2.0 * a * mus
    c0 = a * mus * mus + (jnp.log(pi_norm) - 0.5 * jnp.log(2.0 * math.pi * sig2)) * log2e

    w2 = w * w
    le = a[:, :, None] * w2[None, :, :] + (b[:, :, None] * w[None, :, :] + c0[:, :, None])
    e = jnp.exp2(le)
    s = jnp.sum(e, axis=0)
    c = log2e / (t_ref[0, 0] * (s + EPS))
    p = jnp.exp2(e * c[None, :, :] - 121.0)
    denom = jnp.sum(p, axis=0)
    num = jnp.sum(p * mus[:, :, None], axis=0)
    out_ref[...] = num / denom


def _tc_call(weights, tc_rows, pis, mus, sigmas, temp):
    R, C = weights.shape
    K = pis.shape[0]
    B = 16
    return pl.pallas_call(
        _gmm_tc_body,
        grid=(tc_rows // B,),
        in_specs=[
            pl.BlockSpec((B, C), lambda i: (i, 0)),
            pl.BlockSpec((K, 1), lambda i: (0, 0)),
            pl.BlockSpec((K, 1), lambda i: (0, 0)),
            pl.BlockSpec((K, 1), lambda i: (0, 0)),
            pl.BlockSpec((1, 1), lambda i: (0, 0)),
        ],
        out_specs=pl.BlockSpec((B, C), lambda i: (i, 0)),
        out_shape=jax.ShapeDtypeStruct((tc_rows, C), weights.dtype),
        compiler_params=pltpu.CompilerParams(
            dimension_semantics=("parallel",),
            vmem_limit_bytes=100663296),
    )(weights, pis, mus, sigmas, temp)


def kernel(weights, mu, pi_k, pi_zero, sigma, sigma_zero, temperature):
    K = mu.shape[0] + 1
    R, C = weights.shape
    pis = jnp.concatenate([pi_zero, pi_k]).reshape(K, 1)
    mus = jnp.concatenate([jnp.zeros((1,), weights.dtype), mu]).reshape(K, 1)
    sigmas = jnp.concatenate([sigma_zero, sigma]).reshape(K, 1)
    temp = temperature.reshape(1, 1)

    tc_rows = R - _SC_ROWS
    outs = []
    if tc_rows > 0:
        outs.append(_tc_call(weights, tc_rows, pis, mus, sigmas, temp))
    if _SC_ROWS > 0:
        pi_n = jnp.abs(pis[:, 0])
        pi_n = pi_n / jnp.sum(pi_n)
        sc_sig2 = sigmas[:, 0] * sigmas[:, 0]
        sc_a = -0.5 / sc_sig2
        sc_mus = mus[:, 0]
        sc_b = -2.0 * sc_a * sc_mus
        sc_c0 = sc_a * sc_mus * sc_mus + jnp.log(pi_n) - 0.5 * jnp.log(
            2.0 * math.pi * sc_sig2)
        misc = jnp.zeros((_SC_LANES,), jnp.float32).at[0].set(
            1.0 / temperature[0])
        w_flat = weights.reshape(-1)
        n_sc = _SC_ROWS * C
        sck = _make_sc_kernel(n_sc, tc_rows * C, K)
        out_sc = sck(w_flat, sc_a, sc_b, sc_c0, sc_mus, misc)
        outs.append(out_sc.reshape(_SC_ROWS, C))
    if len(outs) == 1:
        return outs[0]
    return jnp.concatenate(outs, axis=0)
